# baseline (speedup 1.0000x reference)
import functools

import jax
import jax.numpy as jnp
from jax import lax
from jax.experimental import pallas as pl
from jax.experimental.pallas import tpu as pltpu
from jax.experimental.pallas import tpu_sc as plsc

NC = 2
NS = 16
D = 128
P = 10240
ROWS_T = P // NS
K = 80
NBUF = 2
E1T = 10000
E2T = 5120
EJP = NC * NS * E2T
NBLK1 = E1T // K
NBLK2 = E2T // K


def _sc_segment_sums():
    mesh = plsc.VectorSubcoreMesh(
        core_axis_name="c", subcore_axis_name="s", num_cores=NC,
        num_subcores=NS)

    @functools.partial(
        pl.kernel,
        mesh=mesh,
        out_type=(
            jax.ShapeDtypeStruct((NC * P, D), jnp.float32),
            jax.ShapeDtypeStruct((NC * P, D), jnp.float32),
        ),
        scratch_types=[
            pltpu.VMEM_SHARED((P, D), jnp.float32),
            pltpu.VMEM_SHARED((P,), jnp.float32),
            pltpu.VMEM((NBUF, 3, K), jnp.int32),
            pltpu.VMEM((NBUF, K), jnp.float32),
            pltpu.VMEM((NBUF, K, D), jnp.float32),
            pltpu.VMEM((NBUF, K, D), jnp.float32),
            pltpu.VMEM((K,), jnp.float32),
            pltpu.VMEM((K,), jnp.float32),
            [pltpu.SemaphoreType.DMA] * NBUF,
            [pltpu.SemaphoreType.DMA] * NBUF,
        ],
    )
    def sc_pass(pack1, pack2, wpack, emb_h, negrel_h,
                ones_h, z2d, z1d, sio, jpart, acc, deg,
                idx_ring, w_ring, rows_ring, rel_ring, ones_v,
                degc_v, gsems, ssems):
        cid = lax.axis_index("c")
        sid = lax.axis_index("s")
        wid = cid * NS + sid
        rows0 = sid * ROWS_T

        pltpu.sync_copy(ones_h, ones_v)
        pltpu.sync_copy(z2d.at[pl.ds(rows0, ROWS_T)],
                        acc.at[pl.ds(rows0, ROWS_T)])
        pltpu.sync_copy(z1d.at[pl.ds(rows0, ROWS_T)],
                        deg.at[pl.ds(rows0, ROWS_T)])
        plsc.subcore_barrier()

        pbase1 = wid * NBLK1

        def issue1(p, b):
            pltpu.sync_copy(pack1.at[pbase1 + b], idx_ring.at[p])
            pltpu.async_copy(emb_h.at[idx_ring.at[p, 0]],
                             rows_ring.at[p], gsems[p])
            pltpu.async_copy(negrel_h.at[idx_ring.at[p, 1]],
                             rel_ring.at[p], gsems[p])

        def proc1(p):
            pltpu.make_async_copy(z2d.at[pl.ds(0, K)], rows_ring.at[p],
                                  gsems[p]).wait()
            pltpu.make_async_copy(z2d.at[pl.ds(0, K)], rel_ring.at[p],
                                  gsems[p]).wait()
            def addrel(k, c2):
                for j in range(D // 16):
                    sl = pl.ds(j * 16, 16)
                    rows_ring[p, k, sl] = rows_ring[p, k, sl] + rel_ring[p, k, sl]
                return c2

            lax.fori_loop(0, K, addrel, 0)
            s1 = pltpu.async_copy(rows_ring.at[p], acc.at[idx_ring.at[p, 2]],
                                  ssems[p], add=True)
            s3 = pltpu.async_copy(ones_v, deg.at[idx_ring.at[p, 2]],
                                  ssems[p], add=True)
            s1.wait()
            s3.wait()

        for p in range(NBUF):
            issue1(p, p)

        def body1(b2, carry):
            for p in range(NBUF):
                b = b2 * NBUF + p
                proc1(p)
                nb = b + NBUF

                @pl.when(nb < NBLK1)
                def _():
                    issue1(p, nb)

            return carry

        lax.fori_loop(0, NBLK1 // NBUF, body1, 0)
        for p in range(NBLK1 % NBUF):
            proc1(p)
        plsc.subcore_barrier()

        def flushc(c, carry):
            r = rows0 + c * K
            pltpu.sync_copy(acc.at[pl.ds(r, K)], rows_ring.at[0])
            pltpu.sync_copy(deg.at[pl.ds(r, K)], degc_v)

            def normg(g, c2):
                nv = 1.0 / jnp.maximum(degc_v[pl.ds(g * 16, 16)], 1.0)
                for l in range(16):
                    s = nv[l]
                    k = g * 16 + l
                    for j in range(D // 16):
                        sl = pl.ds(j * 16, 16)
                        rows_ring[0, k, sl] = rows_ring[0, k, sl] * s
                return c2

            lax.fori_loop(0, K // 16, normg, 0)
            pltpu.sync_copy(rows_ring.at[0], sio.at[pl.ds(cid * P + r, K)])
            return carry

        lax.fori_loop(0, ROWS_T // K, flushc, 0)
        pltpu.sync_copy(z2d.at[pl.ds(rows0, ROWS_T)],
                        acc.at[pl.ds(rows0, ROWS_T)])
        plsc.subcore_barrier()

        pbase2 = wid * NBLK2

        def issue2(p, b):
            pltpu.sync_copy(pack2.at[pbase2 + b], idx_ring.at[p])
            pltpu.sync_copy(wpack.at[pbase2 + b], w_ring.at[p])
            pltpu.async_copy(emb_h.at[idx_ring.at[p, 0]],
                             rows_ring.at[p], gsems[p])

        def proc2(p):
            pltpu.make_async_copy(z2d.at[pl.ds(0, K)], rows_ring.at[p],
                                  gsems[p]).wait()

            def mulrow(g, c2):
                wv = w_ring[p, pl.ds(g * 16, 16)]
                for l in range(16):
                    s = wv[l]
                    k = g * 16 + l
                    for j in range(D // 16):
                        sl = pl.ds(j * 16, 16)
                        rows_ring[p, k, sl] = rows_ring[p, k, sl] * s
                return c2

            lax.fori_loop(0, K // 16, mulrow, 0)
            pltpu.async_copy(rows_ring.at[p], acc.at[idx_ring.at[p, 1]],
                             ssems[p], add=True).wait()

        for p in range(NBUF):
            issue2(p, p)

        def body2(b2, carry):
            for p in range(NBUF):
                b = b2 * NBUF + p
                proc2(p)
                nb = b + NBUF

                @pl.when(nb < NBLK2)
                def _():
                    issue2(p, nb)

            return carry

        lax.fori_loop(0, NBLK2 // NBUF, body2, 0)
        for p in range(NBLK2 % NBUF):
            proc2(p)
        plsc.subcore_barrier()

        pltpu.sync_copy(acc.at[pl.ds(rows0, ROWS_T)],
                        jpart.at[pl.ds(cid * P + rows0, ROWS_T)])

    return sc_pass


def _tc_dense(emb, sio, jpart, W_in, W_out, W_loop, W_jump, loop_rel, jw):
    R = 1024
    num_e = emb.shape[0]
    grid = (P // R,)
    hi = jax.lax.Precision.HIGHEST

    def body(jw_ref, emb_ref, sin_ref, sout_ref, j0_ref, j1_ref, wi_ref,
             wo_ref, wl_ref, wjm_ref, lr_ref, out_ref):
        acc = jnp.dot(sin_ref[...], wi_ref[...], precision=hi)
        acc = acc + jnp.dot(sout_ref[...], wo_ref[...], precision=hi)
        acc = acc + jnp.dot(emb_ref[...] - lr_ref[...], wl_ref[...],
                            precision=hi)
        emb2 = jnp.tanh(acc * (1.0 / 3.0))
        jr = jnp.dot(j0_ref[...] + j1_ref[...], wjm_ref[...], precision=hi)
        out_ref[...] = emb2 + jw_ref[0] * jr

    blk = lambda im: pl.BlockSpec((R, D), im)
    wspec = pl.BlockSpec((D, D), lambda i: (0, 0))
    return pl.pallas_call(
        body,
        grid=grid,
        in_specs=[
            pl.BlockSpec(memory_space=pltpu.SMEM),
            blk(lambda i: (i, 0)),
            blk(lambda i: (i, 0)),
            blk(lambda i: (i + grid[0], 0)),
            blk(lambda i: (i, 0)),
            blk(lambda i: (i + grid[0], 0)),
            wspec, wspec, wspec, wspec,
            pl.BlockSpec((1, D), lambda i: (0, 0)),
        ],
        out_specs=blk(lambda i: (i, 0)),
        out_shape=jax.ShapeDtypeStruct((num_e, D), jnp.float32),
    )(jw, emb, sio, sio, jpart, jpart, W_in, W_out, W_loop, W_jump,
      loop_rel)


def kernel(t, emb, change, rel_emb, W_in, W_out, W_loop, loop_rel, W_jump,
           jump_weight, edge_w_jump, edge_index, edge_type, edge_id_jump):
    num_e = emb.shape[0]
    n_jump = edge_id_jump.shape[1]

    src_all = edge_index[0]
    dst_all = edge_index[1]
    pack1 = jnp.stack([src_all.reshape(-1, K), edge_type.reshape(-1, K),
                       dst_all.reshape(-1, K)], axis=1)
    npad = EJP - n_jump
    srcj = jnp.pad(edge_id_jump[0], (0, npad))
    dstj = jnp.concatenate(
        [edge_id_jump[1],
         num_e + (jnp.arange(npad, dtype=jnp.int32) % (P - num_e))])
    wj = jnp.pad(edge_w_jump[:, 0], (0, npad))
    pack2 = jnp.stack([srcj.reshape(-1, K), dstj.reshape(-1, K),
                       jnp.zeros_like(srcj).reshape(-1, K)], axis=1)
    wpack = wj.reshape(-1, K)
    negrel = -rel_emb
    ones_h = jnp.ones((K,), jnp.float32)
    z2d = jnp.zeros((P, D), jnp.float32)
    z1d = jnp.zeros((P,), jnp.float32)

    sc = _sc_segment_sums()
    sio, jpart = sc(pack1, pack2, wpack, emb, negrel, ones_h, z2d, z1d)

    dchange = _tc_dense(emb, sio, jpart, W_in, W_out, W_loop, W_jump,
                        loop_rel, jump_weight)
    return (change, dchange)


# speedup vs baseline: 1.0450x; 1.0450x over previous
import functools

import jax
import jax.numpy as jnp
from jax import lax
from jax.experimental import pallas as pl
from jax.experimental.pallas import tpu as pltpu
from jax.experimental.pallas import tpu_sc as plsc

NC = 2
NS = 16
D = 128
P = 10240
ROWS_T = P // NS
K = 80
NBUF = 2
E1T = 10000
E2T = 5120
EJP = NC * NS * E2T
NBLK1 = E1T // K
NBLK2 = E2T // K
NREP = 32


def _sc_segment_sums():
    mesh = plsc.VectorSubcoreMesh(
        core_axis_name="c", subcore_axis_name="s", num_cores=NC,
        num_subcores=NS)

    @functools.partial(
        pl.kernel,
        mesh=mesh,
        out_type=(
            jax.ShapeDtypeStruct((NC * P, D), jnp.float32),
            jax.ShapeDtypeStruct((NC * P, D), jnp.float32),
        ),
        scratch_types=[
            pltpu.VMEM_SHARED((P, D), jnp.float32),
            pltpu.VMEM_SHARED((P,), jnp.float32),
            pltpu.VMEM((NBUF, 3, K), jnp.int32),
            pltpu.VMEM((NBUF, K), jnp.float32),
            pltpu.VMEM((NBUF, K, D), jnp.float32),
            pltpu.VMEM((NBUF, K, D), jnp.float32),
            pltpu.VMEM((K,), jnp.float32),
            pltpu.VMEM((K,), jnp.float32),
            [pltpu.SemaphoreType.DMA] * NBUF,
            [pltpu.SemaphoreType.DMA] * NBUF,
        ],
    )
    def sc_pass(pack1, pack2, wpack, emb_h, negrel_h,
                ones_h, z2d, z1d, sio, jpart, acc, deg,
                idx_ring, w_ring, rows_ring, rel_ring, ones_v,
                degc_v, gsems, ssems):
        cid = lax.axis_index("c")
        sid = lax.axis_index("s")
        wid = cid * NS + sid
        rows0 = sid * ROWS_T

        pltpu.sync_copy(ones_h, ones_v)
        pltpu.sync_copy(z2d.at[pl.ds(rows0, ROWS_T)],
                        acc.at[pl.ds(rows0, ROWS_T)])
        pltpu.sync_copy(z1d.at[pl.ds(rows0, ROWS_T)],
                        deg.at[pl.ds(rows0, ROWS_T)])
        plsc.subcore_barrier()

        pbase1 = wid * NBLK1

        def issue1(p, b):
            pltpu.sync_copy(pack1.at[pbase1 + b], idx_ring.at[p])
            pltpu.async_copy(emb_h.at[idx_ring.at[p, 0]],
                             rows_ring.at[p], gsems[p])
            pltpu.async_copy(negrel_h.at[idx_ring.at[p, 1]],
                             rel_ring.at[p], gsems[p])

        def proc1(p):
            pltpu.make_async_copy(z2d.at[pl.ds(0, K)], rows_ring.at[p],
                                  gsems[p]).wait()
            pltpu.make_async_copy(z2d.at[pl.ds(0, K)], rel_ring.at[p],
                                  gsems[p]).wait()
            def addrel(k, c2):
                for j in range(D // 16):
                    sl = pl.ds(j * 16, 16)
                    rows_ring[p, k, sl] = rows_ring[p, k, sl] + rel_ring[p, k, sl]
                return c2

            lax.fori_loop(0, K, addrel, 0)
            s1 = pltpu.async_copy(rows_ring.at[p], acc.at[idx_ring.at[p, 2]],
                                  ssems[p], add=True)
            s3 = pltpu.async_copy(ones_v, deg.at[idx_ring.at[p, 2]],
                                  ssems[p], add=True)
            s1.wait()
            s3.wait()

        for p in range(NBUF):
            issue1(p, p)

        def body1(b2, carry):
            for p in range(NBUF):
                b = b2 * NBUF + p
                proc1(p)
                nb = b + NBUF

                @pl.when(nb < NBLK1)
                def _():
                    issue1(p, nb)

            return carry

        lax.fori_loop(0, NBLK1 // NBUF, body1, 0)
        for p in range(NBLK1 % NBUF):
            proc1(p)
        plsc.subcore_barrier()

        def flushc(c, carry):
            r = rows0 + c * K
            pltpu.sync_copy(acc.at[pl.ds(r, K)], rows_ring.at[0])
            pltpu.sync_copy(deg.at[pl.ds(r, K)], degc_v)

            def normg(g, c2):
                nv = 1.0 / jnp.maximum(degc_v[pl.ds(g * 16, 16)], 1.0)
                for l in range(16):
                    s = nv[l]
                    k = g * 16 + l
                    for j in range(D // 16):
                        sl = pl.ds(j * 16, 16)
                        rows_ring[0, k, sl] = rows_ring[0, k, sl] * s
                return c2

            lax.fori_loop(0, K // 16, normg, 0)
            pltpu.sync_copy(rows_ring.at[0], sio.at[pl.ds(cid * P + r, K)])
            return carry

        lax.fori_loop(0, ROWS_T // K, flushc, 0)
        pltpu.sync_copy(z2d.at[pl.ds(rows0, ROWS_T)],
                        acc.at[pl.ds(rows0, ROWS_T)])
        plsc.subcore_barrier()

        pbase2 = wid * NBLK2

        def issue2(p, b):
            pltpu.sync_copy(pack2.at[pbase2 + b], idx_ring.at[p])
            pltpu.sync_copy(wpack.at[pbase2 + b], w_ring.at[p])
            pltpu.async_copy(emb_h.at[idx_ring.at[p, 0]],
                             rows_ring.at[p], gsems[p])

        def proc2(p):
            pltpu.make_async_copy(z2d.at[pl.ds(0, K)], rows_ring.at[p],
                                  gsems[p]).wait()

            def mulrow(g, c2):
                wv = w_ring[p, pl.ds(g * 16, 16)]
                for l in range(16):
                    s = wv[l]
                    k = g * 16 + l
                    for j in range(D // 16):
                        sl = pl.ds(j * 16, 16)
                        rows_ring[p, k, sl] = rows_ring[p, k, sl] * s
                return c2

            lax.fori_loop(0, K // 16, mulrow, 0)
            pltpu.async_copy(rows_ring.at[p], acc.at[idx_ring.at[p, 1]],
                             ssems[p], add=True).wait()

        for p in range(NBUF):
            issue2(p, p)

        def body2(b2, carry):
            for p in range(NBUF):
                b = b2 * NBUF + p
                proc2(p)
                nb = b + NBUF

                @pl.when(nb < NBLK2)
                def _():
                    issue2(p, nb)

            return carry

        lax.fori_loop(0, NBLK2 // NBUF, body2, 0)
        for p in range(NBLK2 % NBUF):
            proc2(p)
        plsc.subcore_barrier()

        pltpu.sync_copy(acc.at[pl.ds(rows0, ROWS_T)],
                        jpart.at[pl.ds(cid * P + rows0, ROWS_T)])

    return sc_pass


def _tc_dense(emb, sio, jpart, W_in, W_out, W_loop, W_jump, loop_rel, jw):
    R = 1024
    num_e = emb.shape[0]
    grid = (P // R,)
    hi = jax.lax.Precision.HIGHEST

    def body(jw_ref, emb_ref, sin_ref, sout_ref, j0_ref, j1_ref, wi_ref,
             wo_ref, wl_ref, wjm_ref, lr_ref, out_ref):
        acc = jnp.dot(sin_ref[...], wi_ref[...], precision=hi)
        acc = acc + jnp.dot(sout_ref[...], wo_ref[...], precision=hi)
        acc = acc + jnp.dot(emb_ref[...] - lr_ref[...], wl_ref[...],
                            precision=hi)
        emb2 = jnp.tanh(acc * (1.0 / 3.0))
        jr = jnp.dot(j0_ref[...] + j1_ref[...], wjm_ref[...], precision=hi)
        out_ref[...] = emb2 + jw_ref[0] * jr

    blk = lambda im: pl.BlockSpec((R, D), im)
    wspec = pl.BlockSpec((D, D), lambda i: (0, 0))
    return pl.pallas_call(
        body,
        grid=grid,
        in_specs=[
            pl.BlockSpec(memory_space=pltpu.SMEM),
            blk(lambda i: (i, 0)),
            blk(lambda i: (i, 0)),
            blk(lambda i: (i + grid[0], 0)),
            blk(lambda i: (i, 0)),
            blk(lambda i: (i + grid[0], 0)),
            wspec, wspec, wspec, wspec,
            pl.BlockSpec((1, D), lambda i: (0, 0)),
        ],
        out_specs=blk(lambda i: (i, 0)),
        out_shape=jax.ShapeDtypeStruct((num_e, D), jnp.float32),
    )(jw, emb, sio, sio, jpart, jpart, W_in, W_out, W_loop, W_jump,
      loop_rel)


def kernel(t, emb, change, rel_emb, W_in, W_out, W_loop, loop_rel, W_jump,
           jump_weight, edge_w_jump, edge_index, edge_type, edge_id_jump):
    num_e = emb.shape[0]
    n_jump = edge_id_jump.shape[1]

    src_all = edge_index[0]
    dst_all = edge_index[1]
    nrel = rel_emb.shape[0]
    et_spread = edge_type + nrel * (
        jnp.arange(edge_type.shape[0], dtype=jnp.int32) % NREP)
    pack1 = jnp.stack([src_all.reshape(-1, K), et_spread.reshape(-1, K),
                       dst_all.reshape(-1, K)], axis=1)
    npad = EJP - n_jump
    srcj = jnp.pad(edge_id_jump[0], (0, npad))
    dstj = jnp.concatenate(
        [edge_id_jump[1],
         num_e + (jnp.arange(npad, dtype=jnp.int32) % (P - num_e))])
    wj = jnp.pad(edge_w_jump[:, 0], (0, npad))
    pack2 = jnp.stack([srcj.reshape(-1, K), dstj.reshape(-1, K),
                       jnp.zeros_like(srcj).reshape(-1, K)], axis=1)
    wpack = wj.reshape(-1, K)
    negrel = jnp.tile(-rel_emb, (NREP, 1))
    ones_h = jnp.ones((K,), jnp.float32)
    z2d = jnp.zeros((P, D), jnp.float32)
    z1d = jnp.zeros((P,), jnp.float32)

    sc = _sc_segment_sums()
    sio, jpart = sc(pack1, pack2, wpack, emb, negrel, ones_h, z2d, z1d)

    dchange = _tc_dense(emb, sio, jpart, W_in, W_out, W_loop, W_jump,
                        loop_rel, jump_weight)
    return (change, dchange)
